# baseline (speedup 1.0000x reference)
import functools

import jax
import jax.numpy as jnp
from jax import lax
from jax.experimental import pallas as pl
from jax.experimental.pallas import tpu as pltpu
from jax.experimental.pallas import tpu_sc as plsc

_NUM_PROJS = 16
_HPB = 4


def _pack_body(mat_ref, proj_ref, out_ref):
    x = mat_ref[0]
    p = proj_ref[0]
    y = jax.lax.dot_general(
        p, x, (((1,), (2,)), ((0,), (0,))),
        preferred_element_type=jnp.float32,
    )
    bits = (y > 0).astype(jnp.int32)
    enc = jnp.left_shift(
        jnp.int32(1),
        jax.lax.broadcasted_iota(jnp.int32, (1, _NUM_PROJS, 1), 1),
    )
    out_ref[0] = jnp.sum(bits * enc, axis=1)


def _tc_bucket_ids(mat, proj_dir):
    B, H, S, D = mat.shape
    grid = (B * H) // _HPB
    n_proj_grp = H // _HPB
    mat_r = mat.reshape(grid, _HPB, S, D)
    proj_r = proj_dir.reshape(n_proj_grp, _HPB, D, _NUM_PROJS)
    out = pl.pallas_call(
        _pack_body,
        grid=(grid,),
        in_specs=[
            pl.BlockSpec((1, _HPB, S, D), lambda i: (i, 0, 0, 0)),
            pl.BlockSpec((1, _HPB, D, _NUM_PROJS),
                         lambda i: (i % n_proj_grp, 0, 0, 0)),
        ],
        out_specs=pl.BlockSpec((1, _HPB, S), lambda i: (i, 0, 0)),
        out_shape=jax.ShapeDtypeStruct((grid, _HPB, S), jnp.int32),
    )(mat_r, proj_r)
    return out.reshape(B * H * S)


def _sc_perm_gather(table, idx):
    n = idx.shape[0]
    tbl = table.shape[0]
    info = plsc.get_sparse_core_info()
    nw = info.num_cores * info.num_subcores
    nl = info.num_lanes
    bpw = n // nw
    mesh = plsc.VectorSubcoreMesh(core_axis_name="c", subcore_axis_name="s")

    @functools.partial(
        pl.kernel,
        out_type=jax.ShapeDtypeStruct((n,), jnp.int32),
        mesh=mesh,
        compiler_params=pltpu.CompilerParams(needs_layout_passes=False),
        scratch_types=[
            pltpu.VMEM((tbl,), jnp.int32),
            pltpu.VMEM((bpw,), jnp.int32),
            pltpu.VMEM((bpw,), jnp.int32),
            pltpu.SemaphoreType.DMA,
        ],
    )
    def k(table_hbm, idx_hbm, out_hbm, table_v, idx_v, rows_v, sem):
        wid = lax.axis_index("s") * info.num_cores + lax.axis_index("c")
        base = wid * bpw
        cp = pltpu.async_copy(table_hbm, table_v, sem)
        pltpu.sync_copy(idx_hbm.at[pl.ds(base, bpw)], idx_v)
        cp.wait()

        def body(j, carry):
            iv = idx_v[pl.ds(j * nl, nl)]
            rows_v[pl.ds(j * nl, nl)] = plsc.load_gather(table_v, [iv])
            return carry

        lax.fori_loop(0, bpw // nl, body, 0)
        pltpu.sync_copy(rows_v, out_hbm.at[pl.ds(base, bpw)])

    return k(table, idx)


def kernel(mat, proj_dir):
    B, H, S, _ = mat.shape
    bin_ids = _tc_bucket_ids(mat, proj_dir)
    i = jnp.arange(2 ** _NUM_PROJS, dtype=jnp.int32)
    perm_table = i ^ (i >> 1)
    out = _sc_perm_gather(perm_table, bin_ids)
    return out.reshape(B, H, S)


# speedup vs baseline: 1.9665x; 1.9665x over previous
import jax
import jax.numpy as jnp
from jax import lax
from jax.experimental import pallas as pl
from jax.experimental.pallas import tpu as pltpu

_NUM_PROJS = 16
_NBUF = 6


def _lsh_body(mat_hbm, proj_ref, out_ref, buf, sems):
    g = pl.program_id(0)
    n = pl.num_programs(0)
    H = proj_ref.shape[0]

    def start(c):
        slot = lax.rem(c, _NBUF)
        pltpu.make_async_copy(
            mat_hbm.at[pl.ds(c, 1)], buf.at[pl.ds(slot, 1)], sems.at[slot]
        ).start()

    @pl.when(g == 0)
    def _prime():
        for k in range(_NBUF - 1):
            start(jnp.int32(k))

    @pl.when(g + _NBUF - 1 < n)
    def _ahead():
        start(g + _NBUF - 1)

    slot = lax.rem(g, _NBUF)
    pltpu.make_async_copy(
        mat_hbm.at[pl.ds(g, 1)], buf.at[pl.ds(slot, 1)], sems.at[slot]
    ).wait()

    x = buf[slot]
    p = proj_ref[lax.rem(g, H)]
    y = jax.lax.dot_general(
        p, x, (((0,), (1,)), ((), ())),
        preferred_element_type=jnp.float32,
    )
    bits = (y > 0).astype(jnp.int32)
    enc = jnp.left_shift(
        jnp.int32(1),
        jax.lax.broadcasted_iota(jnp.int32, (_NUM_PROJS, 1), 0),
    )
    b = jnp.sum(bits * enc, axis=0, keepdims=True)
    out_ref[0] = b ^ (b >> 1)


def kernel(mat, proj_dir):
    B, H, S, D = mat.shape
    grid = B * H
    mat_r = mat.reshape(grid, S, D)
    proj_r = proj_dir.reshape(H, D, _NUM_PROJS)
    out = pl.pallas_call(
        _lsh_body,
        grid=(grid,),
        in_specs=[
            pl.BlockSpec(memory_space=pltpu.MemorySpace.HBM),
            pl.BlockSpec(memory_space=pltpu.MemorySpace.VMEM),
        ],
        out_specs=pl.BlockSpec((1, 1, S), lambda i: (i, 0, 0)),
        out_shape=jax.ShapeDtypeStruct((grid, 1, S), jnp.int32),
        scratch_shapes=[
            pltpu.VMEM((_NBUF, S, D), jnp.float32),
            pltpu.SemaphoreType.DMA((_NBUF,)),
        ],
    )(mat_r, proj_r)
    return out.reshape(B, H, S)
